# Initial kernel scaffold; baseline (speedup 1.0000x reference)
import jax
import jax.numpy as jnp
from jax.experimental import pallas as pl


def kernel(x, Wr, Wg, Wu, Wd, Wgs, Wus, Wds):
    raise NotImplementedError("write your pallas kernel here")


# speedup vs baseline: 1.2205x; 1.2205x over previous
import functools

import jax
import jax.numpy as jnp
from jax import lax
from jax.experimental import pallas as pl
from jax.experimental.pallas import tpu as pltpu
from jax.experimental.pallas import tpu_sc as plsc

T = 4096
H = 1024
F = 4096
E = 8
KTOP = 2
G = E + 1
BT = 256
P = T * KTOP + T + G * BT
NB = P // BT
FC = 512
NFC = F // FC

NC, NS = 2, 16
NW = NC * NS
BPW = P // NW
GC = 24
NGC = BPW // GC
RPW = T // NW
CC = 16
NCC = RPW // CC

@functools.lru_cache(maxsize=None)
def _sc_mesh():
    return plsc.VectorSubcoreMesh(core_axis_name="c", subcore_axis_name="s")


def _router_body(x_ref, wr_ref, e_ref, w_ref):
    logits = lax.dot_general(x_ref[...], wr_ref[...], (((1,), (1,)), ((), ())),
                             preferred_element_type=jnp.float32)
    iota = lax.broadcasted_iota(jnp.int32, logits.shape, 1)
    m0 = jnp.max(logits, axis=1, keepdims=True)
    i0 = jnp.min(jnp.where(logits == m0, iota, E), axis=1, keepdims=True)
    masked = jnp.where(iota == i0, -jnp.inf, logits)
    m1 = jnp.max(masked, axis=1, keepdims=True)
    i1 = jnp.min(jnp.where(masked == m1, iota, E), axis=1, keepdims=True)
    w0 = 1.0 / (1.0 + jnp.exp(m1 - m0))
    e_ref[...] = jnp.concatenate([i0, i1], axis=1)
    w_ref[...] = jnp.concatenate([w0, 1.0 - w0], axis=1)


def _router(x, Wr):
    return pl.pallas_call(
        _router_body,
        out_shape=(jax.ShapeDtypeStruct((T, KTOP), jnp.int32),
                   jax.ShapeDtypeStruct((T, KTOP), jnp.float32)),
    )(x, Wr)


def _dispatch(e2, w2):
    ef = jnp.concatenate([e2.reshape(-1), jnp.full((T,), E, jnp.int32)])
    wf = jnp.concatenate([w2.reshape(-1), jnp.ones((T,), jnp.float32)])
    tok = jnp.concatenate([jnp.arange(T * KTOP, dtype=jnp.int32) // KTOP,
                           jnp.arange(T, dtype=jnp.int32)])
    onehot = (ef[:, None] == jnp.arange(G, dtype=jnp.int32)[None, :]).astype(jnp.int32)
    ranks = jnp.cumsum(onehot, axis=0) - 1
    counts = jnp.sum(onehot, axis=0)
    padded = ((counts + BT - 1) // BT) * BT
    offs = jnp.concatenate([jnp.zeros((1,), jnp.int32),
                            jnp.cumsum(padded)])
    dest = offs[ef] + jnp.sum(onehot * ranks, axis=1)
    gtok = jnp.zeros((P,), jnp.int32).at[dest].set(tok)
    gw = jnp.zeros((P,), jnp.float32).at[dest].set(wf)
    nvalid = offs[G] // BT
    bstart = jnp.arange(NB, dtype=jnp.int32) * BT
    be = jnp.searchsorted(offs[:G], bstart, side="right").astype(jnp.int32) - 1
    be = jnp.minimum(be, G - 1)
    dp = dest[: T * KTOP].reshape(T, KTOP)
    d0, d1 = dp[:, 0], dp[:, 1]
    ds = dest[T * KTOP:]
    return gtok, gw, be, nvalid, d0, d1, ds


def _gather_body(x_hbm, gtok_hbm, xs_hbm, idx_v, rows_v):
    wid = lax.axis_index("s") * NC + lax.axis_index("c")
    base = wid * BPW
    pltpu.sync_copy(gtok_hbm.at[pl.ds(base, BPW)], idx_v)
    for k in range(NGC):
        pltpu.sync_copy(x_hbm.at[idx_v.at[pl.ds(k * GC, GC)]], rows_v)
        pltpu.sync_copy(rows_v, xs_hbm.at[pl.ds(base + k * GC, GC)])


@functools.lru_cache(maxsize=None)
def _gather():
    return pl.kernel(
        _gather_body,
        out_type=jax.ShapeDtypeStruct((P, H), jnp.float32),
        mesh=_sc_mesh(),
        scratch_types=[pltpu.VMEM((BPW,), jnp.int32),
                       pltpu.VMEM((GC, H), jnp.float32)],
    )


def _mlp_body(be_ref, nv_ref, xs_ref, wg_ref, wu_ref, wd_ref, gw_ref, y_ref):
    i = pl.program_id(0)

    @pl.when(i < nv_ref[0])
    def _():
        xb = xs_ref[...].astype(jnp.bfloat16)
        acc = jnp.zeros((BT, H), jnp.float32)
        for c in range(NFC):
            wg_c = wg_ref[0, c * FC:(c + 1) * FC, :]
            wu_c = wu_ref[0, c * FC:(c + 1) * FC, :]
            wd_c = wd_ref[0, :, c * FC:(c + 1) * FC]
            g = lax.dot_general(xb, wg_c, (((1,), (1,)), ((), ())),
                                preferred_element_type=jnp.float32)
            u = lax.dot_general(xb, wu_c, (((1,), (1,)), ((), ())),
                                preferred_element_type=jnp.float32)
            hc = (g / (1.0 + jnp.exp(-g)) * u).astype(jnp.bfloat16)
            acc = acc + lax.dot_general(hc, wd_c, (((1,), (1,)), ((), ())),
                                        preferred_element_type=jnp.float32)
        y_ref[...] = acc * gw_ref[0]


def _grouped_mlp(be, nvalid, xs, WgA, WuA, WdA, gw3):
    grid_spec = pltpu.PrefetchScalarGridSpec(
        num_scalar_prefetch=2,
        grid=(NB,),
        in_specs=[
            pl.BlockSpec((BT, H), lambda i, be, nv: (i, 0)),
            pl.BlockSpec((1, F, H), lambda i, be, nv: (be[i], 0, 0)),
            pl.BlockSpec((1, F, H), lambda i, be, nv: (be[i], 0, 0)),
            pl.BlockSpec((1, H, F), lambda i, be, nv: (be[i], 0, 0)),
            pl.BlockSpec((1, BT, 1), lambda i, be, nv: (i, 0, 0)),
        ],
        out_specs=pl.BlockSpec((BT, H), lambda i, be, nv: (i, 0)),
    )
    return pl.pallas_call(
        _mlp_body,
        grid_spec=grid_spec,
        out_shape=jax.ShapeDtypeStruct((P, H), jnp.float32),
        compiler_params=pltpu.CompilerParams(
            dimension_semantics=("arbitrary",)),
    )(be, nvalid, xs, WgA, WuA, WdA, gw3)


def _combine_body(y_hbm, d0_hbm, d1_hbm, ds_hbm, out_hbm,
                  i0_v, i1_v, is_v, a_v, b_v, c_v, o_v):
    wid = lax.axis_index("s") * NC + lax.axis_index("c")
    base = wid * RPW
    pltpu.sync_copy(d0_hbm.at[pl.ds(base, RPW)], i0_v)
    pltpu.sync_copy(d1_hbm.at[pl.ds(base, RPW)], i1_v)
    pltpu.sync_copy(ds_hbm.at[pl.ds(base, RPW)], is_v)
    for k in range(NCC):
        pltpu.sync_copy(y_hbm.at[i0_v.at[pl.ds(k * CC, CC)]], a_v)
        pltpu.sync_copy(y_hbm.at[i1_v.at[pl.ds(k * CC, CC)]], b_v)
        pltpu.sync_copy(y_hbm.at[is_v.at[pl.ds(k * CC, CC)]], c_v)

        def body(r, _):
            def inner(c, _):
                sl = pl.ds(c * 16, 16)
                o_v[r, sl] = a_v[r, sl] + b_v[r, sl] + c_v[r, sl]
                return 0
            return lax.fori_loop(0, H // 16, inner, 0)
        lax.fori_loop(0, CC, body, 0)
        pltpu.sync_copy(o_v, out_hbm.at[pl.ds(base + k * CC, CC)])


@functools.lru_cache(maxsize=None)
def _combine():
    return pl.kernel(
        _combine_body,
        out_type=jax.ShapeDtypeStruct((T, H), jnp.float32),
        mesh=_sc_mesh(),
        scratch_types=[pltpu.VMEM((RPW,), jnp.int32),
                       pltpu.VMEM((RPW,), jnp.int32),
                       pltpu.VMEM((RPW,), jnp.int32),
                       pltpu.VMEM((CC, H), jnp.float32),
                       pltpu.VMEM((CC, H), jnp.float32),
                       pltpu.VMEM((CC, H), jnp.float32),
                       pltpu.VMEM((CC, H), jnp.float32)],
    )


def kernel(x, Wr, Wg, Wu, Wd, Wgs, Wus, Wds):
    x2 = x.reshape(T, H)
    WgA = jnp.concatenate([Wg, Wgs[None]], axis=0).astype(jnp.bfloat16)
    WuA = jnp.concatenate([Wu, Wus[None]], axis=0).astype(jnp.bfloat16)
    WdA = jnp.concatenate([Wd, Wds[None]], axis=0).astype(jnp.bfloat16)

    e2, w2 = _router(x2, Wr)
    gtok, gw, be, nvalid, d0, d1, ds = _dispatch(e2, w2)

    xs = _gather()(x2, gtok)
    gw3 = gw.reshape(NB, BT, 1)
    y = _grouped_mlp(be, nvalid.reshape(1), xs, WgA, WuA, WdA, gw3)
    out = _combine()(y, d0, d1, ds)
    return out.reshape(x.shape)
